# Initial kernel scaffold; baseline (speedup 1.0000x reference)
import jax
import jax.numpy as jnp
from jax.experimental import pallas as pl


def kernel(x_p, x_np, y, edge_index_p, edge_index_np, W_gcn, b_gcn, W1, b1, W2, b2):
    raise NotImplementedError("write your pallas kernel here")


# speedup vs baseline: 6.6945x; 6.6945x over previous
import functools

import jax
import jax.numpy as jnp
from jax import lax
from jax.experimental import pallas as pl
from jax.experimental.pallas import tpu as pltpu
from jax.experimental.pallas import tpu_sc as plsc

N_P = 32768
N_NP = 4096
D = 128
B = 8
E_P = 524288
E_NP = 65536
WALK_LEN = 7

C = 4
CW = 32
G = 128
TILES = 16
RB = 2048

GP_P = E_P // G
GP_NP = E_NP // G
PGRP_P = GP_P // TILES
PGRP_NP = GP_NP // TILES
SUB = 32


def _deg_body(dst_p, dst_np, zeros, ones,
              deg_p, deg_np,
              acc_p, acc_np, dstv, ones_v):
    c = lax.axis_index("c")
    s = lax.axis_index("s")
    pltpu.sync_copy(ones, ones_v)
    pltpu.sync_copy(zeros.at[pl.ds(0, N_P // TILES)],
                    acc_p.at[pl.ds(s * (N_P // TILES), N_P // TILES)])
    pltpu.sync_copy(zeros.at[pl.ds(0, N_NP // TILES)],
                    acc_np.at[pl.ds(s * (N_NP // TILES), N_NP // TILES)])
    plsc.subcore_barrier()

    @pl.when(c == 0)
    def _():
        pltpu.sync_copy(dst_p.at[pl.ds(s * PGRP_P, PGRP_P)],
                        dstv.at[pl.ds(0, PGRP_P)])

        def body(j, carry):
            pltpu.sync_copy(ones_v, acc_p.at[dstv.at[j]], add=True)
            return carry
        lax.fori_loop(0, PGRP_P, body, 0)

    @pl.when(c == 1)
    def _():
        pltpu.sync_copy(dst_np.at[pl.ds(s * PGRP_NP, PGRP_NP)],
                        dstv.at[pl.ds(0, PGRP_NP)])

        def body(j, carry):
            pltpu.sync_copy(ones_v, acc_np.at[dstv.at[j]], add=True)
            return carry
        lax.fori_loop(0, PGRP_NP, body, 0)

    plsc.subcore_barrier()

    @pl.when(c == 0)
    def _():
        pltpu.sync_copy(acc_p.at[pl.ds(s * (N_P // TILES), N_P // TILES)],
                        deg_p.at[pl.ds(s * (N_P // TILES), N_P // TILES)])

    @pl.when(c == 1)
    def _():
        pltpu.sync_copy(acc_np.at[pl.ds(s * (N_NP // TILES), N_NP // TILES)],
                        deg_np.at[pl.ds(s * (N_NP // TILES), N_NP // TILES)])


def _spmm_body(gs_p, gidx_p, dst_p, gs_np, gidx_np, dst_np,
               agg_p, agg_np,
               acc_p, acc_np, rows, gidxv, dstv, gidxnv, dstnv):
    c = lax.axis_index("c")
    s = lax.axis_index("s")
    npt = N_P // TILES
    nnt = N_NP // TILES

    for q in range(2):
        chunk = c * 2 + q
        pltpu.sync_copy(gs_p.at[pl.ds(chunk * N_P + s * npt, npt)],
                        acc_p.at[pl.ds(s * npt, npt)])
        pltpu.sync_copy(gs_np.at[pl.ds(chunk * N_NP + s * nnt, nnt)],
                        acc_np.at[pl.ds(s * nnt, nnt)])
        plsc.subcore_barrier()

        def sbody(sj, carry):
            base = s * PGRP_P + sj * SUB
            pltpu.sync_copy(gidx_p.at[chunk, pl.ds(base, SUB)], gidxv)
            pltpu.sync_copy(dst_p.at[pl.ds(base, SUB)], dstv)

            def pbody(j, carry2):
                pltpu.sync_copy(gs_p.at[gidxv.at[j]], rows)
                pltpu.sync_copy(rows, acc_p.at[dstv.at[j]], add=True)
                return carry2
            return lax.fori_loop(0, SUB, pbody, carry)
        lax.fori_loop(0, PGRP_P // SUB, sbody, 0)

        pltpu.sync_copy(gidx_np.at[chunk, pl.ds(s * PGRP_NP, PGRP_NP)], gidxnv)
        pltpu.sync_copy(dst_np.at[pl.ds(s * PGRP_NP, PGRP_NP)], dstnv)

        def npbody(j, carry):
            pltpu.sync_copy(gs_np.at[gidxnv.at[j]], rows)
            pltpu.sync_copy(rows, acc_np.at[dstnv.at[j]], add=True)
            return carry
        lax.fori_loop(0, PGRP_NP, npbody, 0)

        plsc.subcore_barrier()
        pltpu.sync_copy(acc_p.at[pl.ds(s * npt, npt)],
                        agg_p.at[pl.ds(chunk * N_P + s * npt, npt)])
        pltpu.sync_copy(acc_np.at[pl.ds(s * nnt, nnt)],
                        agg_np.at[pl.ds(chunk * N_NP + s * nnt, nnt)])


_SC_MESH = plsc.VectorSubcoreMesh(core_axis_name="c", subcore_axis_name="s")
_SC_PARAMS = pltpu.CompilerParams(use_tc_tiling_on_sc=False)

_deg_call = pl.kernel(
    _deg_body,
    out_type=(jax.ShapeDtypeStruct((N_P, 16), jnp.float32),
              jax.ShapeDtypeStruct((N_NP, 16), jnp.float32)),
    mesh=_SC_MESH,
    scratch_types=[
        pltpu.VMEM_SHARED((N_P, 16), jnp.float32),
        pltpu.VMEM_SHARED((N_NP, 16), jnp.float32),
        pltpu.VMEM((PGRP_P, G), jnp.int32),
        pltpu.VMEM((G, 16), jnp.float32),
    ],
    compiler_params=_SC_PARAMS,
)

_spmm_call = pl.kernel(
    _spmm_body,
    out_type=(jax.ShapeDtypeStruct((C * N_P, CW), jnp.float32),
              jax.ShapeDtypeStruct((C * N_NP, CW), jnp.float32)),
    mesh=_SC_MESH,
    scratch_types=[
        pltpu.VMEM_SHARED((N_P, CW), jnp.float32),
        pltpu.VMEM_SHARED((N_NP, CW), jnp.float32),
        pltpu.VMEM((G, CW), jnp.float32),
        pltpu.VMEM((SUB, G), jnp.int32),
        pltpu.VMEM((SUB, G), jnp.int32),
        pltpu.VMEM((PGRP_NP, G), jnp.int32),
        pltpu.VMEM((PGRP_NP, G), jnp.int32),
    ],
    compiler_params=_SC_PARAMS,
)


def _prep_body(x_ref, w_ref, deg_ref, out_ref):
    dinv = lax.rsqrt(deg_ref[:, 0:1] + 1.0)
    g = jnp.dot(x_ref[:, :], w_ref[:, :], preferred_element_type=jnp.float32)
    gs = g * dinv
    for cc in range(C):
        out_ref[cc, :, :] = gs[:, cc * CW:(cc + 1) * CW]


def _hop_body(agg_ref, deg_ref, b_ref, w_ref, out_ref, tr_ref):
    j = pl.program_id(0)
    dinv = lax.rsqrt(deg_ref[:, 0:1] + 1.0)
    rows = agg_ref.shape[1]
    rowi = lax.broadcasted_iota(jnp.int32, (rows, CW), 0)
    coli = lax.broadcasted_iota(jnp.int32, (rows, CW), 1)
    acc = jnp.zeros((rows, D), dtype=jnp.float32)
    tr = jnp.float32(0.0)
    for cc in range(C):
        h_cc = agg_ref[cc] * dinv + b_ref[:, cc * CW:(cc + 1) * CW]
        tr = tr + jnp.sum(jnp.where(rowi == coli + cc * CW, h_cc, 0.0))
        acc = acc + jnp.dot(h_cc, w_ref[cc * CW:(cc + 1) * CW, :],
                            preferred_element_type=jnp.float32)
    gs = acc * dinv
    for cc in range(C):
        out_ref[cc, :, :] = gs[:, cc * CW:(cc + 1) * CW]

    @pl.when(j % 2 == 0)
    def _():
        tr_ref[...] = jnp.reshape(tr, (1, 1, 1))


def _trace_only_body(agg_ref, deg_ref, b_ref, tr_ref):
    j = pl.program_id(0)
    dinv = lax.rsqrt(deg_ref[:, 0:1] + 1.0)
    rows = agg_ref.shape[1]
    rowi = lax.broadcasted_iota(jnp.int32, (rows, CW), 0)
    coli = lax.broadcasted_iota(jnp.int32, (rows, CW), 1)
    tr = jnp.float32(0.0)
    for cc in range(C):
        h_cc = agg_ref[cc] * dinv + b_ref[:, cc * CW:(cc + 1) * CW]
        tr = tr + jnp.sum(jnp.where(rowi == coli + cc * CW, h_cc, 0.0))

    @pl.when(j % 2 == 0)
    def _():
        tr_ref[...] = jnp.reshape(tr, (1, 1, 1))


def _head_body(tp_ref, tnp_ref, y_ref, w1_ref, b1_ref, w2_ref, b2_ref, z_ref):
    p = (tp_ref[:, :] - tnp_ref[:, :]) * ((y_ref[:, :] - 0.5) * 2.0)
    mu = jnp.mean(p, axis=0, keepdims=True)
    var = jnp.sum((p - mu) ** 2, axis=0, keepdims=True) / (B - 1)
    p = (p - mu) / jnp.sqrt(var)
    h = jnp.maximum(
        jnp.dot(p, w1_ref[:, :], preferred_element_type=jnp.float32)
        + b1_ref[:, :], 0.0)
    z = (jnp.dot(h, w2_ref[:, :], preferred_element_type=jnp.float32)
         + b2_ref[:, :])
    z_ref[:, :] = 1.0 / (1.0 + jnp.exp(-z))


def _make_prep(n):
    grid = n // RB
    return pl.pallas_call(
        _prep_body,
        grid=(grid,),
        in_specs=[
            pl.BlockSpec((RB, D), lambda j: (j, 0)),
            pl.BlockSpec((D, D), lambda j: (0, 0)),
            pl.BlockSpec((RB, 16), lambda j: (j, 0)),
        ],
        out_specs=pl.BlockSpec((C, RB, CW), lambda j: (0, j, 0)),
        out_shape=jax.ShapeDtypeStruct((C, n, CW), jnp.float32),
    )


def _make_hop(n):
    grid = n // RB
    return pl.pallas_call(
        _hop_body,
        grid=(grid,),
        in_specs=[
            pl.BlockSpec((C, RB, CW), lambda j: (0, j, 0)),
            pl.BlockSpec((RB, 16), lambda j: (j, 0)),
            pl.BlockSpec((1, D), lambda j: (0, 0)),
            pl.BlockSpec((D, D), lambda j: (0, 0)),
        ],
        out_specs=[
            pl.BlockSpec((C, RB, CW), lambda j: (0, j, 0)),
            pl.BlockSpec((1, 1, 1), lambda j: (j // 2, 0, 0)),
        ],
        out_shape=[
            jax.ShapeDtypeStruct((C, n, CW), jnp.float32),
            jax.ShapeDtypeStruct((max(n // RB // 2, 1), 1, 1), jnp.float32),
        ],
    )


def _make_trace_only(n):
    grid = n // RB
    return pl.pallas_call(
        _trace_only_body,
        grid=(grid,),
        in_specs=[
            pl.BlockSpec((C, RB, CW), lambda j: (0, j, 0)),
            pl.BlockSpec((RB, 16), lambda j: (j, 0)),
            pl.BlockSpec((1, D), lambda j: (0, 0)),
        ],
        out_specs=pl.BlockSpec((1, 1, 1), lambda j: (j // 2, 0, 0)),
        out_shape=jax.ShapeDtypeStruct((max(n // RB // 2, 1), 1, 1),
                                       jnp.float32),
    )


_head_call = pl.pallas_call(
    _head_body,
    out_shape=jax.ShapeDtypeStruct((B, 1), jnp.float32),
)


def kernel(x_p, x_np, y, edge_index_p, edge_index_np, W_gcn, b_gcn,
           W1, b1, W2, b2):
    src_p = edge_index_p[0]
    dst_p = edge_index_p[1]
    src_np = edge_index_np[0]
    dst_np = edge_index_np[1]
    offs_p = (jnp.arange(C, dtype=jnp.int32) * N_P)[:, None]
    offs_np = (jnp.arange(C, dtype=jnp.int32) * N_NP)[:, None]
    gidx_p = (src_p[None, :] + offs_p).reshape(C, GP_P, G)
    gidx_np = (src_np[None, :] + offs_np).reshape(C, GP_NP, G)
    dstm_p = dst_p.reshape(GP_P, G)
    dstm_np = dst_np.reshape(GP_NP, G)
    zeros = jnp.zeros((N_P // TILES, 16), jnp.float32)
    ones = jnp.ones((G, 16), jnp.float32)
    b2d = b_gcn.reshape(1, D)

    deg_p, deg_np = _deg_call(dstm_p, dstm_np, zeros, ones)

    prep_p = _make_prep(N_P)
    prep_np = _make_prep(N_NP)
    hop_p = _make_hop(N_P)
    hop_np = _make_hop(N_NP)
    trace_p = _make_trace_only(N_P)
    trace_np = _make_trace_only(N_NP)

    gs_p = prep_p(x_p, W_gcn, deg_p).reshape(C * N_P, CW)
    gs_np = prep_np(x_np, W_gcn, deg_np).reshape(C * N_NP, CW)

    trs_p = []
    trs_np = []
    for k in range(WALK_LEN):
        agg_p, agg_np = _spmm_call(gs_p, gidx_p, dstm_p,
                                   gs_np, gidx_np, dstm_np)
        agg_p4 = agg_p.reshape(C, N_P, CW)
        agg_np4 = agg_np.reshape(C, N_NP, CW)
        if k < WALK_LEN - 1:
            gs_p, tp = hop_p(agg_p4, deg_p, b2d, W_gcn)
            gs_np, tn = hop_np(agg_np4, deg_np, b2d, W_gcn)
            gs_p = gs_p.reshape(C * N_P, CW)
            gs_np = gs_np.reshape(C * N_NP, CW)
        else:
            tp = trace_p(agg_p4, deg_p, b2d)
            tn = trace_np(agg_np4, deg_np, b2d)
        trs_p.append(tp.reshape(B))
        trs_np.append(tn.reshape(1))

    tp_all = jnp.stack(trs_p, axis=1)
    tnp_all = jnp.stack(trs_np, axis=1)
    z = _head_call(tp_all, tnp_all, y.astype(jnp.float32),
                   W1, b1.reshape(1, 15), W2, b2.reshape(1, 1))
    return z


# speedup vs baseline: 12.7195x; 1.9000x over previous
import functools

import jax
import jax.numpy as jnp
from jax import lax
from jax.experimental import pallas as pl
from jax.experimental.pallas import tpu as pltpu
from jax.experimental.pallas import tpu_sc as plsc

N_P = 32768
N_NP = 4096
D = 128
B = 8
E_P = 524288
E_NP = 65536
WALK_LEN = 7

C = 4
CW = 32
G = 128
TILES = 16
RB = 2048

GP_P = E_P // G
GP_NP = E_NP // G
PGRP_P = GP_P // TILES
PGRP_NP = GP_NP // TILES
SUBP = 64
NBUF = 4


def _deg_body(dst_p, dst_np, zeros, ones,
              deg_p, deg_np,
              acc_p, acc_np, dstv, ones_v, dsem):
    c = lax.axis_index("c")
    s = lax.axis_index("s")
    pltpu.sync_copy(ones, ones_v)
    pltpu.sync_copy(zeros.at[pl.ds(0, N_P // TILES)],
                    acc_p.at[pl.ds(s * (N_P // TILES), N_P // TILES)])
    pltpu.sync_copy(zeros.at[pl.ds(0, N_NP // TILES)],
                    acc_np.at[pl.ds(s * (N_NP // TILES), N_NP // TILES)])
    plsc.subcore_barrier()

    @pl.when(c == 0)
    def _():
        pltpu.sync_copy(dst_p.at[pl.ds(s * PGRP_P, PGRP_P)],
                        dstv.at[pl.ds(0, PGRP_P)])

        def body(j, carry):
            pltpu.async_copy(ones_v, acc_p.at[dstv.at[j]], dsem, add=True)
            return carry
        lax.fori_loop(0, PGRP_P, body, 0)

        def drain(j, carry):
            pltpu.make_async_copy(ones_v, acc_p.at[dstv.at[0]], dsem).wait()
            return carry
        lax.fori_loop(0, PGRP_P, drain, 0)

    @pl.when(c == 1)
    def _():
        pltpu.sync_copy(dst_np.at[pl.ds(s * PGRP_NP, PGRP_NP)],
                        dstv.at[pl.ds(0, PGRP_NP)])

        def body(j, carry):
            pltpu.async_copy(ones_v, acc_np.at[dstv.at[j]], dsem, add=True)
            return carry
        lax.fori_loop(0, PGRP_NP, body, 0)

        def drain(j, carry):
            pltpu.make_async_copy(ones_v, acc_np.at[dstv.at[0]], dsem).wait()
            return carry
        lax.fori_loop(0, PGRP_NP, drain, 0)

    plsc.subcore_barrier()

    @pl.when(c == 0)
    def _():
        pltpu.sync_copy(acc_p.at[pl.ds(s * (N_P // TILES), N_P // TILES)],
                        deg_p.at[pl.ds(s * (N_P // TILES), N_P // TILES)])

    @pl.when(c == 1)
    def _():
        pltpu.sync_copy(acc_np.at[pl.ds(s * (N_NP // TILES), N_NP // TILES)],
                        deg_np.at[pl.ds(s * (N_NP // TILES), N_NP // TILES)])


def _spmm_body(gs_p, gidx_p, dst_p, gs_np, gidx_np, dst_np,
               agg_p, agg_np,
               acc_p, acc_np, rows, gidxv, dstv, gidxnv, dstnv, gsem, ssem):
    c = lax.axis_index("c")
    s = lax.axis_index("s")
    npt = N_P // TILES
    nnt = N_NP // TILES

    def run_stream(table, accr, gv, dv, nsub):
        for b in range(NBUF):
            pltpu.async_copy(table.at[gv.at[b]], rows.at[b], gsem.at[b])

        def it_body(it, carry):
            for b in range(NBUF):
                lj = it * NBUF + b
                pltpu.make_async_copy(table.at[gv.at[lj]], rows.at[b],
                                      gsem.at[b]).wait()
                pltpu.async_copy(rows.at[b], accr.at[dv.at[lj]], ssem.at[b],
                                 add=True)
            for b in range(NBUF):
                lj2 = (it + 1) * NBUF + b

                @pl.when(lj2 < nsub)
                def _():
                    pltpu.make_async_copy(rows.at[b], accr.at[dv.at[b]],
                                          ssem.at[b]).wait()
                    pltpu.async_copy(table.at[gv.at[lj2]], rows.at[b],
                                     gsem.at[b])
            return carry
        lax.fori_loop(0, nsub // NBUF, it_body, 0)
        for b in range(NBUF):
            pltpu.make_async_copy(rows.at[b], accr.at[dv.at[b]],
                                  ssem.at[b]).wait()

    for q in range(2):
        chunk = c * 2 + q
        pltpu.sync_copy(gs_p.at[pl.ds(chunk * N_P + s * npt, npt)],
                        acc_p.at[pl.ds(s * npt, npt)])
        pltpu.sync_copy(gs_np.at[pl.ds(chunk * N_NP + s * nnt, nnt)],
                        acc_np.at[pl.ds(s * nnt, nnt)])
        plsc.subcore_barrier()

        def sbody(sj, carry):
            base = s * PGRP_P + sj * SUBP
            pltpu.sync_copy(gidx_p.at[chunk, pl.ds(base, SUBP)], gidxv)
            pltpu.sync_copy(dst_p.at[pl.ds(base, SUBP)], dstv)
            run_stream(gs_p, acc_p, gidxv, dstv, SUBP)
            return carry
        lax.fori_loop(0, PGRP_P // SUBP, sbody, 0)

        pltpu.sync_copy(gidx_np.at[chunk, pl.ds(s * PGRP_NP, PGRP_NP)], gidxnv)
        pltpu.sync_copy(dst_np.at[pl.ds(s * PGRP_NP, PGRP_NP)], dstnv)
        run_stream(gs_np, acc_np, gidxnv, dstnv, PGRP_NP)

        plsc.subcore_barrier()
        pltpu.sync_copy(acc_p.at[pl.ds(s * npt, npt)],
                        agg_p.at[pl.ds(chunk * N_P + s * npt, npt)])
        pltpu.sync_copy(acc_np.at[pl.ds(s * nnt, nnt)],
                        agg_np.at[pl.ds(chunk * N_NP + s * nnt, nnt)])


_SC_MESH = plsc.VectorSubcoreMesh(core_axis_name="c", subcore_axis_name="s")
_SC_PARAMS = pltpu.CompilerParams(use_tc_tiling_on_sc=False)

_deg_call = pl.kernel(
    _deg_body,
    out_type=(jax.ShapeDtypeStruct((N_P, 16), jnp.float32),
              jax.ShapeDtypeStruct((N_NP, 16), jnp.float32)),
    mesh=_SC_MESH,
    scratch_types=[
        pltpu.VMEM_SHARED((N_P, 16), jnp.float32),
        pltpu.VMEM_SHARED((N_NP, 16), jnp.float32),
        pltpu.VMEM((PGRP_P, G), jnp.int32),
        pltpu.VMEM((G, 16), jnp.float32),
        pltpu.SemaphoreType.DMA,
    ],
    compiler_params=_SC_PARAMS,
)

_spmm_call = pl.kernel(
    _spmm_body,
    out_type=(jax.ShapeDtypeStruct((C * N_P, CW), jnp.float32),
              jax.ShapeDtypeStruct((C * N_NP, CW), jnp.float32)),
    mesh=_SC_MESH,
    scratch_types=[
        pltpu.VMEM_SHARED((N_P, CW), jnp.float32),
        pltpu.VMEM_SHARED((N_NP, CW), jnp.float32),
        pltpu.VMEM((NBUF, G, CW), jnp.float32),
        pltpu.VMEM((SUBP, G), jnp.int32),
        pltpu.VMEM((SUBP, G), jnp.int32),
        pltpu.VMEM((PGRP_NP, G), jnp.int32),
        pltpu.VMEM((PGRP_NP, G), jnp.int32),
        pltpu.SemaphoreType.DMA((NBUF,)),
        pltpu.SemaphoreType.DMA((NBUF,)),
    ],
    compiler_params=_SC_PARAMS,
)


def _prep_body(x_ref, w_ref, deg_ref, out_ref):
    dinv = lax.rsqrt(deg_ref[:, 0:1] + 1.0)
    g = jnp.dot(x_ref[:, :], w_ref[:, :], preferred_element_type=jnp.float32)
    gs = g * dinv
    for cc in range(C):
        out_ref[cc, :, :] = gs[:, cc * CW:(cc + 1) * CW]


def _hop_body(agg_ref, deg_ref, b_ref, w_ref, out_ref, tr_ref):
    j = pl.program_id(0)
    dinv = lax.rsqrt(deg_ref[:, 0:1] + 1.0)
    rows = agg_ref.shape[1]
    rowi = lax.broadcasted_iota(jnp.int32, (rows, CW), 0)
    coli = lax.broadcasted_iota(jnp.int32, (rows, CW), 1)
    acc = jnp.zeros((rows, D), dtype=jnp.float32)
    tr = jnp.float32(0.0)
    for cc in range(C):
        h_cc = agg_ref[cc] * dinv + b_ref[:, cc * CW:(cc + 1) * CW]
        tr = tr + jnp.sum(jnp.where(rowi == coli + cc * CW, h_cc, 0.0))
        acc = acc + jnp.dot(h_cc, w_ref[cc * CW:(cc + 1) * CW, :],
                            preferred_element_type=jnp.float32)
    gs = acc * dinv
    for cc in range(C):
        out_ref[cc, :, :] = gs[:, cc * CW:(cc + 1) * CW]

    @pl.when(j % 2 == 0)
    def _():
        tr_ref[...] = jnp.reshape(tr, (1, 1, 1))


def _trace_only_body(agg_ref, deg_ref, b_ref, tr_ref):
    j = pl.program_id(0)
    dinv = lax.rsqrt(deg_ref[:, 0:1] + 1.0)
    rows = agg_ref.shape[1]
    rowi = lax.broadcasted_iota(jnp.int32, (rows, CW), 0)
    coli = lax.broadcasted_iota(jnp.int32, (rows, CW), 1)
    tr = jnp.float32(0.0)
    for cc in range(C):
        h_cc = agg_ref[cc] * dinv + b_ref[:, cc * CW:(cc + 1) * CW]
        tr = tr + jnp.sum(jnp.where(rowi == coli + cc * CW, h_cc, 0.0))

    @pl.when(j % 2 == 0)
    def _():
        tr_ref[...] = jnp.reshape(tr, (1, 1, 1))


def _head_body(tp_ref, tnp_ref, y_ref, w1_ref, b1_ref, w2_ref, b2_ref, z_ref):
    p = (tp_ref[:, :] - tnp_ref[:, :]) * ((y_ref[:, :] - 0.5) * 2.0)
    mu = jnp.mean(p, axis=0, keepdims=True)
    var = jnp.sum((p - mu) ** 2, axis=0, keepdims=True) / (B - 1)
    p = (p - mu) / jnp.sqrt(var)
    h = jnp.maximum(
        jnp.dot(p, w1_ref[:, :], preferred_element_type=jnp.float32)
        + b1_ref[:, :], 0.0)
    z = (jnp.dot(h, w2_ref[:, :], preferred_element_type=jnp.float32)
         + b2_ref[:, :])
    z_ref[:, :] = 1.0 / (1.0 + jnp.exp(-z))


def _make_prep(n):
    grid = n // RB
    return pl.pallas_call(
        _prep_body,
        grid=(grid,),
        in_specs=[
            pl.BlockSpec((RB, D), lambda j: (j, 0)),
            pl.BlockSpec((D, D), lambda j: (0, 0)),
            pl.BlockSpec((RB, 16), lambda j: (j, 0)),
        ],
        out_specs=pl.BlockSpec((C, RB, CW), lambda j: (0, j, 0)),
        out_shape=jax.ShapeDtypeStruct((C, n, CW), jnp.float32),
    )


def _make_hop(n):
    grid = n // RB
    return pl.pallas_call(
        _hop_body,
        grid=(grid,),
        in_specs=[
            pl.BlockSpec((C, RB, CW), lambda j: (0, j, 0)),
            pl.BlockSpec((RB, 16), lambda j: (j, 0)),
            pl.BlockSpec((1, D), lambda j: (0, 0)),
            pl.BlockSpec((D, D), lambda j: (0, 0)),
        ],
        out_specs=[
            pl.BlockSpec((C, RB, CW), lambda j: (0, j, 0)),
            pl.BlockSpec((1, 1, 1), lambda j: (j // 2, 0, 0)),
        ],
        out_shape=[
            jax.ShapeDtypeStruct((C, n, CW), jnp.float32),
            jax.ShapeDtypeStruct((max(n // RB // 2, 1), 1, 1), jnp.float32),
        ],
    )


def _make_trace_only(n):
    grid = n // RB
    return pl.pallas_call(
        _trace_only_body,
        grid=(grid,),
        in_specs=[
            pl.BlockSpec((C, RB, CW), lambda j: (0, j, 0)),
            pl.BlockSpec((RB, 16), lambda j: (j, 0)),
            pl.BlockSpec((1, D), lambda j: (0, 0)),
        ],
        out_specs=pl.BlockSpec((1, 1, 1), lambda j: (j // 2, 0, 0)),
        out_shape=jax.ShapeDtypeStruct((max(n // RB // 2, 1), 1, 1),
                                       jnp.float32),
    )


_head_call = pl.pallas_call(
    _head_body,
    out_shape=jax.ShapeDtypeStruct((B, 1), jnp.float32),
)


def kernel(x_p, x_np, y, edge_index_p, edge_index_np, W_gcn, b_gcn,
           W1, b1, W2, b2):
    src_p = edge_index_p[0]
    dst_p = edge_index_p[1]
    src_np = edge_index_np[0]
    dst_np = edge_index_np[1]
    offs_p = (jnp.arange(C, dtype=jnp.int32) * N_P)[:, None]
    offs_np = (jnp.arange(C, dtype=jnp.int32) * N_NP)[:, None]
    gidx_p = (src_p[None, :] + offs_p).reshape(C, GP_P, G)
    gidx_np = (src_np[None, :] + offs_np).reshape(C, GP_NP, G)
    dstm_p = dst_p.reshape(GP_P, G)
    dstm_np = dst_np.reshape(GP_NP, G)
    zeros = jnp.zeros((N_P // TILES, 16), jnp.float32)
    ones = jnp.ones((G, 16), jnp.float32)
    b2d = b_gcn.reshape(1, D)

    deg_p, deg_np = _deg_call(dstm_p, dstm_np, zeros, ones)

    prep_p = _make_prep(N_P)
    prep_np = _make_prep(N_NP)
    hop_p = _make_hop(N_P)
    hop_np = _make_hop(N_NP)
    trace_p = _make_trace_only(N_P)
    trace_np = _make_trace_only(N_NP)

    gs_p = prep_p(x_p, W_gcn, deg_p).reshape(C * N_P, CW)
    gs_np = prep_np(x_np, W_gcn, deg_np).reshape(C * N_NP, CW)

    trs_p = []
    trs_np = []
    for k in range(WALK_LEN):
        agg_p, agg_np = _spmm_call(gs_p, gidx_p, dstm_p,
                                   gs_np, gidx_np, dstm_np)
        agg_p4 = agg_p.reshape(C, N_P, CW)
        agg_np4 = agg_np.reshape(C, N_NP, CW)
        if k < WALK_LEN - 1:
            gs_p, tp = hop_p(agg_p4, deg_p, b2d, W_gcn)
            gs_np, tn = hop_np(agg_np4, deg_np, b2d, W_gcn)
            gs_p = gs_p.reshape(C * N_P, CW)
            gs_np = gs_np.reshape(C * N_NP, CW)
        else:
            tp = trace_p(agg_p4, deg_p, b2d)
            tn = trace_np(agg_np4, deg_np, b2d)
        trs_p.append(tp.reshape(B))
        trs_np.append(tn.reshape(1))

    tp_all = jnp.stack(trs_p, axis=1)
    tnp_all = jnp.stack(trs_np, axis=1)
    z = _head_call(tp_all, tnp_all, y.astype(jnp.float32),
                   W1, b1.reshape(1, 15), W2, b2.reshape(1, 1))
    return z


# speedup vs baseline: 17.4839x; 1.3746x over previous
import jax
import jax.numpy as jnp
from jax import lax
from jax.experimental import pallas as pl
from jax.experimental.pallas import tpu as pltpu
from jax.experimental.pallas import tpu_sc as plsc

N_P = 32768
N_NP = 4096
D = 128
B = 8
E_P = 524288
E_NP = 65536
WALK_LEN = 7

C = 4
CW = 32
G = 128
TILES = 16
RB = 2048

GP_P = E_P // G
GP_NP = E_NP // G
PGRP_P = GP_P // TILES
PGRP_NP = GP_NP // TILES
SUBP = 64
NBUF = 4


def _deg_body(dst_p, dst_np, zeros, ones,
              deg_p, deg_np,
              acc_p, acc_np, dstv, ones_v, dsem):
    c = lax.axis_index("c")
    s = lax.axis_index("s")
    pltpu.sync_copy(ones, ones_v)
    pltpu.sync_copy(zeros.at[pl.ds(0, N_P // TILES)],
                    acc_p.at[pl.ds(s * (N_P // TILES), N_P // TILES)])
    pltpu.sync_copy(zeros.at[pl.ds(0, N_NP // TILES)],
                    acc_np.at[pl.ds(s * (N_NP // TILES), N_NP // TILES)])
    plsc.subcore_barrier()

    @pl.when(c == 0)
    def _():
        pltpu.sync_copy(dst_p.at[pl.ds(s * PGRP_P, PGRP_P)],
                        dstv.at[pl.ds(0, PGRP_P)])

        def body(j, carry):
            pltpu.async_copy(ones_v, acc_p.at[dstv.at[j]], dsem, add=True)
            return carry
        lax.fori_loop(0, PGRP_P, body, 0)

        def drain(j, carry):
            pltpu.make_async_copy(ones_v, acc_p.at[dstv.at[0]], dsem).wait()
            return carry
        lax.fori_loop(0, PGRP_P, drain, 0)

    @pl.when(c == 1)
    def _():
        pltpu.sync_copy(dst_np.at[pl.ds(s * PGRP_NP, PGRP_NP)],
                        dstv.at[pl.ds(0, PGRP_NP)])

        def body(j, carry):
            pltpu.async_copy(ones_v, acc_np.at[dstv.at[j]], dsem, add=True)
            return carry
        lax.fori_loop(0, PGRP_NP, body, 0)

        def drain(j, carry):
            pltpu.make_async_copy(ones_v, acc_np.at[dstv.at[0]], dsem).wait()
            return carry
        lax.fori_loop(0, PGRP_NP, drain, 0)

    plsc.subcore_barrier()

    @pl.when(c == 0)
    def _():
        pltpu.sync_copy(acc_p.at[pl.ds(s * (N_P // TILES), N_P // TILES)],
                        deg_p.at[pl.ds(s * (N_P // TILES), N_P // TILES)])

    @pl.when(c == 1)
    def _():
        pltpu.sync_copy(acc_np.at[pl.ds(s * (N_NP // TILES), N_NP // TILES)],
                        deg_np.at[pl.ds(s * (N_NP // TILES), N_NP // TILES)])


def _spmm_body(gs_p, gidx_p, dst_p, gs_np, gidx_np, dst_np,
               agg_p, agg_np, tbl_p, tbl_np,
               acc_p, acc_np, rows, gidxv, dstv, gidxnv, dstnv,
               gsem, ssem, tsem):
    c = lax.axis_index("c")
    s = lax.axis_index("s")
    npt = N_P // TILES
    nnt = N_NP // TILES

    def stage(gs, tbl, accr, nt, chunk, col):
        npieces = nt // G

        def src(pc):
            return gs.at[pl.ds(s * nt + pc * G, G), pl.ds(col, CW)]

        for pc in range(min(NBUF, npieces)):
            pltpu.async_copy(src(pc), rows.at[pc], gsem.at[pc])
        for pc in range(npieces):
            b = pc % NBUF
            pltpu.make_async_copy(src(pc), rows.at[b], gsem.at[b]).wait()
            pltpu.async_copy(
                rows.at[b],
                tbl.at[pl.ds(chunk * nt * TILES + s * nt + pc * G, G)],
                ssem.at[b])
            pltpu.async_copy(rows.at[b], accr.at[pl.ds(s * nt + pc * G, G)],
                             tsem.at[b])
            nxt = pc + NBUF
            if nxt < npieces:
                pltpu.make_async_copy(rows.at[b], tbl.at[pl.ds(0, G)],
                                      ssem.at[b]).wait()
                pltpu.make_async_copy(rows.at[b], accr.at[pl.ds(0, G)],
                                      tsem.at[b]).wait()
                pltpu.async_copy(src(nxt), rows.at[b], gsem.at[b])
        for pc in range(max(0, npieces - NBUF), npieces):
            b = pc % NBUF
            pltpu.make_async_copy(rows.at[b], tbl.at[pl.ds(0, G)],
                                  ssem.at[b]).wait()
            pltpu.make_async_copy(rows.at[b], accr.at[pl.ds(0, G)],
                                  tsem.at[b]).wait()

    def run_stream(table, accr, gv, dv, nsub):
        for b in range(NBUF):
            pltpu.async_copy(table.at[gv.at[b]], rows.at[b], gsem.at[b])

        def it_body(it, carry):
            for b in range(NBUF):
                lj = it * NBUF + b
                pltpu.make_async_copy(table.at[gv.at[lj]], rows.at[b],
                                      gsem.at[b]).wait()
                pltpu.async_copy(rows.at[b], accr.at[dv.at[lj]], ssem.at[b],
                                 add=True)
            for b in range(NBUF):
                lj2 = (it + 1) * NBUF + b

                @pl.when(lj2 < nsub)
                def _():
                    pltpu.make_async_copy(rows.at[b], accr.at[dv.at[b]],
                                          ssem.at[b]).wait()
                    pltpu.async_copy(table.at[gv.at[lj2]], rows.at[b],
                                     gsem.at[b])
            return carry
        lax.fori_loop(0, nsub // NBUF, it_body, 0)
        for b in range(NBUF):
            pltpu.make_async_copy(rows.at[b], accr.at[dv.at[b]],
                                  ssem.at[b]).wait()

    pltpu.sync_copy(dst_np.at[pl.ds(s * PGRP_NP, PGRP_NP)], dstnv)

    for q in range(2):
        for cc in range(2):
            @pl.when(c == cc)
            def _(cc=cc, q=q):
                chunk = cc * 2 + q
                col = chunk * CW
                stage(gs_p, tbl_p, acc_p, npt, chunk, col)
                stage(gs_np, tbl_np, acc_np, nnt, chunk, col)
                pltpu.sync_copy(
                    gidx_np.at[chunk, pl.ds(s * PGRP_NP, PGRP_NP)], gidxnv)
        plsc.subcore_barrier()

        for cc in range(2):
            @pl.when(c == cc)
            def _(cc=cc, q=q):
                chunk = cc * 2 + q

                def sbody(sj, carry):
                    base = s * PGRP_P + sj * SUBP
                    pltpu.sync_copy(gidx_p.at[chunk, pl.ds(base, SUBP)],
                                    gidxv)
                    pltpu.sync_copy(dst_p.at[pl.ds(base, SUBP)], dstv)
                    run_stream(tbl_p, acc_p, gidxv, dstv, SUBP)
                    return carry
                lax.fori_loop(0, PGRP_P // SUBP, sbody, 0)
                run_stream(tbl_np, acc_np, gidxnv, dstnv, PGRP_NP)
        plsc.subcore_barrier()

        for cc in range(2):
            @pl.when(c == cc)
            def _(cc=cc, q=q):
                col = (cc * 2 + q) * CW
                pltpu.sync_copy(acc_p.at[pl.ds(s * npt, npt)],
                                agg_p.at[pl.ds(s * npt, npt), pl.ds(col, CW)])
                pltpu.sync_copy(acc_np.at[pl.ds(s * nnt, nnt)],
                                agg_np.at[pl.ds(s * nnt, nnt), pl.ds(col, CW)])


_SC_MESH = plsc.VectorSubcoreMesh(core_axis_name="c", subcore_axis_name="s")
_SC_PARAMS = pltpu.CompilerParams(use_tc_tiling_on_sc=False)

_deg_call = pl.kernel(
    _deg_body,
    out_type=(jax.ShapeDtypeStruct((N_P, 16), jnp.float32),
              jax.ShapeDtypeStruct((N_NP, 16), jnp.float32)),
    mesh=_SC_MESH,
    scratch_types=[
        pltpu.VMEM_SHARED((N_P, 16), jnp.float32),
        pltpu.VMEM_SHARED((N_NP, 16), jnp.float32),
        pltpu.VMEM((PGRP_P, G), jnp.int32),
        pltpu.VMEM((G, 16), jnp.float32),
        pltpu.SemaphoreType.DMA,
    ],
    compiler_params=_SC_PARAMS,
)

_spmm_call = pl.kernel(
    _spmm_body,
    out_type=(jax.ShapeDtypeStruct((N_P, D), jnp.float32),
              jax.ShapeDtypeStruct((N_NP, D), jnp.float32),
              jax.ShapeDtypeStruct((C * N_P, CW), jnp.float32),
              jax.ShapeDtypeStruct((C * N_NP, CW), jnp.float32)),
    mesh=_SC_MESH,
    scratch_types=[
        pltpu.VMEM_SHARED((N_P, CW), jnp.float32),
        pltpu.VMEM_SHARED((N_NP, CW), jnp.float32),
        pltpu.VMEM((NBUF, G, CW), jnp.float32),
        pltpu.VMEM((SUBP, G), jnp.int32),
        pltpu.VMEM((SUBP, G), jnp.int32),
        pltpu.VMEM((PGRP_NP, G), jnp.int32),
        pltpu.VMEM((PGRP_NP, G), jnp.int32),
        pltpu.SemaphoreType.DMA((NBUF,)),
        pltpu.SemaphoreType.DMA((NBUF,)),
        pltpu.SemaphoreType.DMA((NBUF,)),
    ],
    compiler_params=_SC_PARAMS,
)


def _prep_body(x_ref, w_ref, deg_ref, out_ref):
    dinv = lax.rsqrt(deg_ref[:, 0:1] + 1.0)
    g = jnp.dot(x_ref[:, :], w_ref[:, :], preferred_element_type=jnp.float32)
    out_ref[:, :] = g * dinv


def _hop_body(agg_ref, deg_ref, b_ref, w_ref, out_ref, tr_ref):
    j = pl.program_id(0)
    dinv = lax.rsqrt(deg_ref[:, 0:1] + 1.0)
    h = agg_ref[:, :] * dinv + b_ref[:, :]
    rowi = lax.broadcasted_iota(jnp.int32, (RB, D), 0)
    coli = lax.broadcasted_iota(jnp.int32, (RB, D), 1)
    tr = jnp.sum(jnp.where(rowi == coli, h, 0.0))
    g = jnp.dot(h, w_ref[:, :], preferred_element_type=jnp.float32)
    out_ref[:, :] = g * dinv

    @pl.when(j % 2 == 0)
    def _():
        tr_ref[...] = jnp.reshape(tr, (1, 1, 1))


def _trace_only_body(agg_ref, deg_ref, b_ref, tr_ref):
    j = pl.program_id(0)
    dinv = lax.rsqrt(deg_ref[:, 0:1] + 1.0)
    h = agg_ref[:, :] * dinv + b_ref[:, :]
    rowi = lax.broadcasted_iota(jnp.int32, (RB, D), 0)
    coli = lax.broadcasted_iota(jnp.int32, (RB, D), 1)
    tr = jnp.sum(jnp.where(rowi == coli, h, 0.0))

    @pl.when(j % 2 == 0)
    def _():
        tr_ref[...] = jnp.reshape(tr, (1, 1, 1))


def _head_body(tp_ref, tnp_ref, y_ref, w1_ref, b1_ref, w2_ref, b2_ref, z_ref):
    p = (tp_ref[:, :] - tnp_ref[:, :]) * ((y_ref[:, :] - 0.5) * 2.0)
    mu = jnp.mean(p, axis=0, keepdims=True)
    var = jnp.sum((p - mu) ** 2, axis=0, keepdims=True) / (B - 1)
    p = (p - mu) / jnp.sqrt(var)
    h = jnp.maximum(
        jnp.dot(p, w1_ref[:, :], preferred_element_type=jnp.float32)
        + b1_ref[:, :], 0.0)
    z = (jnp.dot(h, w2_ref[:, :], preferred_element_type=jnp.float32)
         + b2_ref[:, :])
    z_ref[:, :] = 1.0 / (1.0 + jnp.exp(-z))


def _make_prep(n):
    return pl.pallas_call(
        _prep_body,
        grid=(n // RB,),
        in_specs=[
            pl.BlockSpec((RB, D), lambda j: (j, 0)),
            pl.BlockSpec((D, D), lambda j: (0, 0)),
            pl.BlockSpec((RB, 16), lambda j: (j, 0)),
        ],
        out_specs=pl.BlockSpec((RB, D), lambda j: (j, 0)),
        out_shape=jax.ShapeDtypeStruct((n, D), jnp.float32),
    )


def _make_hop(n):
    return pl.pallas_call(
        _hop_body,
        grid=(n // RB,),
        in_specs=[
            pl.BlockSpec((RB, D), lambda j: (j, 0)),
            pl.BlockSpec((RB, 16), lambda j: (j, 0)),
            pl.BlockSpec((1, D), lambda j: (0, 0)),
            pl.BlockSpec((D, D), lambda j: (0, 0)),
        ],
        out_specs=[
            pl.BlockSpec((RB, D), lambda j: (j, 0)),
            pl.BlockSpec((1, 1, 1), lambda j: (j // 2, 0, 0)),
        ],
        out_shape=[
            jax.ShapeDtypeStruct((n, D), jnp.float32),
            jax.ShapeDtypeStruct((max(n // RB // 2, 1), 1, 1), jnp.float32),
        ],
    )


def _make_trace_only(n):
    return pl.pallas_call(
        _trace_only_body,
        grid=(n // RB,),
        in_specs=[
            pl.BlockSpec((RB, D), lambda j: (j, 0)),
            pl.BlockSpec((RB, 16), lambda j: (j, 0)),
            pl.BlockSpec((1, D), lambda j: (0, 0)),
        ],
        out_specs=pl.BlockSpec((1, 1, 1), lambda j: (j // 2, 0, 0)),
        out_shape=jax.ShapeDtypeStruct((max(n // RB // 2, 1), 1, 1),
                                       jnp.float32),
    )


_head_call = pl.pallas_call(
    _head_body,
    out_shape=jax.ShapeDtypeStruct((B, 1), jnp.float32),
)


def kernel(x_p, x_np, y, edge_index_p, edge_index_np, W_gcn, b_gcn,
           W1, b1, W2, b2):
    offs_p = (jnp.arange(C, dtype=jnp.int32) * N_P)[:, None]
    offs_np = (jnp.arange(C, dtype=jnp.int32) * N_NP)[:, None]
    gidx_p = (edge_index_p[0][None, :] + offs_p).reshape(C, GP_P, G)
    gidx_np = (edge_index_np[0][None, :] + offs_np).reshape(C, GP_NP, G)
    dstm_p = edge_index_p[1].reshape(GP_P, G)
    dstm_np = edge_index_np[1].reshape(GP_NP, G)
    zeros = jnp.zeros((N_P // TILES, 16), jnp.float32)
    ones = jnp.ones((G, 16), jnp.float32)
    b2d = b_gcn.reshape(1, D)

    deg_p, deg_np = _deg_call(dstm_p, dstm_np, zeros, ones)

    prep_p = _make_prep(N_P)
    prep_np = _make_prep(N_NP)
    hop_p = _make_hop(N_P)
    hop_np = _make_hop(N_NP)
    trace_p = _make_trace_only(N_P)
    trace_np = _make_trace_only(N_NP)

    gs_p = prep_p(x_p, W_gcn, deg_p)
    gs_np = prep_np(x_np, W_gcn, deg_np)

    trs_p = []
    trs_np = []
    for k in range(WALK_LEN):
        agg_p, agg_np, _, _ = _spmm_call(gs_p, gidx_p, dstm_p,
                                         gs_np, gidx_np, dstm_np)
        if k < WALK_LEN - 1:
            gs_p, tp = hop_p(agg_p, deg_p, b2d, W_gcn)
            gs_np, tn = hop_np(agg_np, deg_np, b2d, W_gcn)
        else:
            tp = trace_p(agg_p, deg_p, b2d)
            tn = trace_np(agg_np, deg_np, b2d)
        trs_p.append(tp.reshape(B))
        trs_np.append(tn.reshape(1))

    tp_all = jnp.stack(trs_p, axis=1)
    tnp_all = jnp.stack(trs_np, axis=1)
    z = _head_call(tp_all, tnp_all, y.astype(jnp.float32),
                   W1, b1.reshape(1, 15), W2, b2.reshape(1, 1))
    return z


# speedup vs baseline: 18.1753x; 1.0395x over previous
import jax
import jax.numpy as jnp
from jax import lax
from jax.experimental import pallas as pl
from jax.experimental.pallas import tpu as pltpu
from jax.experimental.pallas import tpu_sc as plsc

N_P = 32768
N_NP = 4096
D = 128
B = 8
E_P = 524288
E_NP = 65536
WALK_LEN = 7

C = 4
CW = 32
G = 128
TILES = 16
RB = 2048

GP_P = E_P // G
GP_NP = E_NP // G
PGRP_P = GP_P // TILES
PGRP_NP = GP_NP // TILES
SUBP = 16
NBUF = 8


def _deg_body(dst_p, dst_np, zeros, ones,
              deg_p, deg_np,
              acc_p, acc_np, dstv, ones_v, dsem):
    c = lax.axis_index("c")
    s = lax.axis_index("s")
    pltpu.sync_copy(ones, ones_v)
    pltpu.sync_copy(zeros.at[pl.ds(0, N_P // TILES)],
                    acc_p.at[pl.ds(s * (N_P // TILES), N_P // TILES)])
    pltpu.sync_copy(zeros.at[pl.ds(0, N_NP // TILES)],
                    acc_np.at[pl.ds(s * (N_NP // TILES), N_NP // TILES)])
    plsc.subcore_barrier()

    @pl.when(c == 0)
    def _():
        pltpu.sync_copy(dst_p.at[pl.ds(s * PGRP_P, PGRP_P)],
                        dstv.at[pl.ds(0, PGRP_P)])

        def body(j, carry):
            pltpu.async_copy(ones_v, acc_p.at[dstv.at[j]], dsem, add=True)
            return carry
        lax.fori_loop(0, PGRP_P, body, 0)

        def drain(j, carry):
            pltpu.make_async_copy(ones_v, acc_p.at[dstv.at[0]], dsem).wait()
            return carry
        lax.fori_loop(0, PGRP_P, drain, 0)

    @pl.when(c == 1)
    def _():
        pltpu.sync_copy(dst_np.at[pl.ds(s * PGRP_NP, PGRP_NP)],
                        dstv.at[pl.ds(0, PGRP_NP)])

        def body(j, carry):
            pltpu.async_copy(ones_v, acc_np.at[dstv.at[j]], dsem, add=True)
            return carry
        lax.fori_loop(0, PGRP_NP, body, 0)

        def drain(j, carry):
            pltpu.make_async_copy(ones_v, acc_np.at[dstv.at[0]], dsem).wait()
            return carry
        lax.fori_loop(0, PGRP_NP, drain, 0)

    plsc.subcore_barrier()

    @pl.when(c == 0)
    def _():
        pltpu.sync_copy(acc_p.at[pl.ds(s * (N_P // TILES), N_P // TILES)],
                        deg_p.at[pl.ds(s * (N_P // TILES), N_P // TILES)])

    @pl.when(c == 1)
    def _():
        pltpu.sync_copy(acc_np.at[pl.ds(s * (N_NP // TILES), N_NP // TILES)],
                        deg_np.at[pl.ds(s * (N_NP // TILES), N_NP // TILES)])


def _spmm_body(gs_p, gidx_p, dst_p, gs_np, gidx_np, dst_np,
               agg_p, agg_np, tbl_p, tbl_np,
               acc_p, acc_np, rows, gidxv, dstv, gidxnv, dstnv,
               gsem, ssem, tsem, isem, isem2):
    c = lax.axis_index("c")
    s = lax.axis_index("s")
    npt = N_P // TILES
    nnt = N_NP // TILES

    def stage(gs, tbl, accr, nt, chunk, col):
        npieces = nt // G

        def src(pc):
            return gs.at[pl.ds(s * nt + pc * G, G), pl.ds(col, CW)]

        for pc in range(min(NBUF, npieces)):
            pltpu.async_copy(src(pc), rows.at[pc], gsem.at[pc])
        for pc in range(npieces):
            b = pc % NBUF
            pltpu.make_async_copy(src(pc), rows.at[b], gsem.at[b]).wait()
            pltpu.async_copy(
                rows.at[b],
                tbl.at[pl.ds(chunk * nt * TILES + s * nt + pc * G, G)],
                ssem.at[b])
            pltpu.async_copy(rows.at[b], accr.at[pl.ds(s * nt + pc * G, G)],
                             tsem.at[b])
            nxt = pc + NBUF
            if nxt < npieces:
                pltpu.make_async_copy(rows.at[b], tbl.at[pl.ds(0, G)],
                                      ssem.at[b]).wait()
                pltpu.make_async_copy(rows.at[b], accr.at[pl.ds(0, G)],
                                      tsem.at[b]).wait()
                pltpu.async_copy(src(nxt), rows.at[b], gsem.at[b])
        for pc in range(max(0, npieces - NBUF), npieces):
            b = pc % NBUF
            pltpu.make_async_copy(rows.at[b], tbl.at[pl.ds(0, G)],
                                  ssem.at[b]).wait()
            pltpu.make_async_copy(rows.at[b], accr.at[pl.ds(0, G)],
                                  tsem.at[b]).wait()

    def run_stream(table, accr, gv, dv, nsub):
        for b in range(NBUF):
            pltpu.async_copy(table.at[gv.at[b]], rows.at[b], gsem.at[b])

        def it_body(it, carry):
            for b in range(NBUF):
                lj = it * NBUF + b
                pltpu.make_async_copy(table.at[gv.at[lj]], rows.at[b],
                                      gsem.at[b]).wait()
                pltpu.async_copy(rows.at[b], accr.at[dv.at[lj]], ssem.at[b],
                                 add=True)
            for b in range(NBUF):
                lj2 = (it + 1) * NBUF + b

                @pl.when(lj2 < nsub)
                def _():
                    pltpu.make_async_copy(rows.at[b], accr.at[dv.at[b]],
                                          ssem.at[b]).wait()
                    pltpu.async_copy(table.at[gv.at[lj2]], rows.at[b],
                                     gsem.at[b])
            return carry
        lax.fori_loop(0, nsub // NBUF, it_body, 0)
        for b in range(NBUF):
            pltpu.make_async_copy(rows.at[b], accr.at[dv.at[b]],
                                  ssem.at[b]).wait()

    pltpu.sync_copy(dst_np.at[pl.ds(s * PGRP_NP, PGRP_NP)], dstnv)

    for q in range(2):
        for cc in range(2):
            @pl.when(c == cc)
            def _(cc=cc, q=q):
                chunk = cc * 2 + q
                col = chunk * CW
                stage(gs_p, tbl_p, acc_p, npt, chunk, col)
                stage(gs_np, tbl_np, acc_np, nnt, chunk, col)
                pltpu.sync_copy(
                    gidx_np.at[chunk, pl.ds(s * PGRP_NP, PGRP_NP)], gidxnv)
        plsc.subcore_barrier()

        for cc in range(2):
            @pl.when(c == cc)
            def _(cc=cc, q=q):
                chunk = cc * 2 + q
                nsb = PGRP_P // SUBP

                def iload(sj, slot):
                    base = s * PGRP_P + sj * SUBP
                    pltpu.async_copy(gidx_p.at[chunk, pl.ds(base, SUBP)],
                                     gidxv.at[slot], isem)
                    pltpu.async_copy(dst_p.at[pl.ds(base, SUBP)],
                                     dstv.at[slot], isem2)

                def iwait(slot):
                    pltpu.make_async_copy(dst_p.at[pl.ds(0, SUBP)],
                                          gidxv.at[slot], isem).wait()
                    pltpu.make_async_copy(dst_p.at[pl.ds(0, SUBP)],
                                          dstv.at[slot], isem2).wait()

                iload(0, 0)

                def sbody(h, carry):
                    sj0 = h * 2
                    iwait(0)

                    @pl.when(sj0 + 1 < nsb)
                    def _():
                        iload(sj0 + 1, 1)
                    run_stream(tbl_p, acc_p, gidxv.at[0], dstv.at[0], SUBP)
                    iwait(1)

                    @pl.when(sj0 + 2 < nsb)
                    def _():
                        iload(sj0 + 2, 0)
                    run_stream(tbl_p, acc_p, gidxv.at[1], dstv.at[1], SUBP)
                    return carry
                lax.fori_loop(0, nsb // 2, sbody, 0)
                run_stream(tbl_np, acc_np, gidxnv, dstnv, PGRP_NP)
        plsc.subcore_barrier()

        for cc in range(2):
            @pl.when(c == cc)
            def _(cc=cc, q=q):
                col = (cc * 2 + q) * CW
                pltpu.sync_copy(acc_p.at[pl.ds(s * npt, npt)],
                                agg_p.at[pl.ds(s * npt, npt), pl.ds(col, CW)])
                pltpu.sync_copy(acc_np.at[pl.ds(s * nnt, nnt)],
                                agg_np.at[pl.ds(s * nnt, nnt), pl.ds(col, CW)])


_SC_MESH = plsc.VectorSubcoreMesh(core_axis_name="c", subcore_axis_name="s")
_SC_PARAMS = pltpu.CompilerParams(use_tc_tiling_on_sc=False)

_deg_call = pl.kernel(
    _deg_body,
    out_type=(jax.ShapeDtypeStruct((N_P, 16), jnp.float32),
              jax.ShapeDtypeStruct((N_NP, 16), jnp.float32)),
    mesh=_SC_MESH,
    scratch_types=[
        pltpu.VMEM_SHARED((N_P, 16), jnp.float32),
        pltpu.VMEM_SHARED((N_NP, 16), jnp.float32),
        pltpu.VMEM((PGRP_P, G), jnp.int32),
        pltpu.VMEM((G, 16), jnp.float32),
        pltpu.SemaphoreType.DMA,
    ],
    compiler_params=_SC_PARAMS,
)

_spmm_call = pl.kernel(
    _spmm_body,
    out_type=(jax.ShapeDtypeStruct((N_P, D), jnp.float32),
              jax.ShapeDtypeStruct((N_NP, D), jnp.float32),
              jax.ShapeDtypeStruct((C * N_P, CW), jnp.float32),
              jax.ShapeDtypeStruct((C * N_NP, CW), jnp.float32)),
    mesh=_SC_MESH,
    scratch_types=[
        pltpu.VMEM_SHARED((N_P, CW), jnp.float32),
        pltpu.VMEM_SHARED((N_NP, CW), jnp.float32),
        pltpu.VMEM((NBUF, G, CW), jnp.float32),
        pltpu.VMEM((2, SUBP, G), jnp.int32),
        pltpu.VMEM((2, SUBP, G), jnp.int32),
        pltpu.VMEM((PGRP_NP, G), jnp.int32),
        pltpu.VMEM((PGRP_NP, G), jnp.int32),
        pltpu.SemaphoreType.DMA((NBUF,)),
        pltpu.SemaphoreType.DMA((NBUF,)),
        pltpu.SemaphoreType.DMA((NBUF,)),
        pltpu.SemaphoreType.DMA,
        pltpu.SemaphoreType.DMA,
    ],
    compiler_params=_SC_PARAMS,
)


def _prep_body(x_ref, w_ref, deg_ref, out_ref):
    dinv = lax.rsqrt(deg_ref[:, 0:1] + 1.0)
    g = jnp.dot(x_ref[:, :], w_ref[:, :], preferred_element_type=jnp.float32)
    out_ref[:, :] = g * dinv


def _hop_body(agg_ref, deg_ref, b_ref, w_ref, out_ref, tr_ref):
    j = pl.program_id(0)
    dinv = lax.rsqrt(deg_ref[:, 0:1] + 1.0)
    h = agg_ref[:, :] * dinv + b_ref[:, :]
    rowi = lax.broadcasted_iota(jnp.int32, (RB, D), 0)
    coli = lax.broadcasted_iota(jnp.int32, (RB, D), 1)
    tr = jnp.sum(jnp.where(rowi == coli, h, 0.0))
    g = jnp.dot(h, w_ref[:, :], preferred_element_type=jnp.float32)
    out_ref[:, :] = g * dinv

    @pl.when(j % 2 == 0)
    def _():
        tr_ref[...] = jnp.reshape(tr, (1, 1, 1))


def _trace_only_body(agg_ref, deg_ref, b_ref, tr_ref):
    j = pl.program_id(0)
    dinv = lax.rsqrt(deg_ref[:, 0:1] + 1.0)
    h = agg_ref[:, :] * dinv + b_ref[:, :]
    rowi = lax.broadcasted_iota(jnp.int32, (RB, D), 0)
    coli = lax.broadcasted_iota(jnp.int32, (RB, D), 1)
    tr = jnp.sum(jnp.where(rowi == coli, h, 0.0))

    @pl.when(j % 2 == 0)
    def _():
        tr_ref[...] = jnp.reshape(tr, (1, 1, 1))


def _head_body(tp_ref, tnp_ref, y_ref, w1_ref, b1_ref, w2_ref, b2_ref, z_ref):
    p = (tp_ref[:, :] - tnp_ref[:, :]) * ((y_ref[:, :] - 0.5) * 2.0)
    mu = jnp.mean(p, axis=0, keepdims=True)
    var = jnp.sum((p - mu) ** 2, axis=0, keepdims=True) / (B - 1)
    p = (p - mu) / jnp.sqrt(var)
    h = jnp.maximum(
        jnp.dot(p, w1_ref[:, :], preferred_element_type=jnp.float32)
        + b1_ref[:, :], 0.0)
    z = (jnp.dot(h, w2_ref[:, :], preferred_element_type=jnp.float32)
         + b2_ref[:, :])
    z_ref[:, :] = 1.0 / (1.0 + jnp.exp(-z))


def _make_prep(n):
    return pl.pallas_call(
        _prep_body,
        grid=(n // RB,),
        in_specs=[
            pl.BlockSpec((RB, D), lambda j: (j, 0)),
            pl.BlockSpec((D, D), lambda j: (0, 0)),
            pl.BlockSpec((RB, 16), lambda j: (j, 0)),
        ],
        out_specs=pl.BlockSpec((RB, D), lambda j: (j, 0)),
        out_shape=jax.ShapeDtypeStruct((n, D), jnp.float32),
    )


def _make_hop(n):
    return pl.pallas_call(
        _hop_body,
        grid=(n // RB,),
        in_specs=[
            pl.BlockSpec((RB, D), lambda j: (j, 0)),
            pl.BlockSpec((RB, 16), lambda j: (j, 0)),
            pl.BlockSpec((1, D), lambda j: (0, 0)),
            pl.BlockSpec((D, D), lambda j: (0, 0)),
        ],
        out_specs=[
            pl.BlockSpec((RB, D), lambda j: (j, 0)),
            pl.BlockSpec((1, 1, 1), lambda j: (j // 2, 0, 0)),
        ],
        out_shape=[
            jax.ShapeDtypeStruct((n, D), jnp.float32),
            jax.ShapeDtypeStruct((max(n // RB // 2, 1), 1, 1), jnp.float32),
        ],
    )


def _make_trace_only(n):
    return pl.pallas_call(
        _trace_only_body,
        grid=(n // RB,),
        in_specs=[
            pl.BlockSpec((RB, D), lambda j: (j, 0)),
            pl.BlockSpec((RB, 16), lambda j: (j, 0)),
            pl.BlockSpec((1, D), lambda j: (0, 0)),
        ],
        out_specs=pl.BlockSpec((1, 1, 1), lambda j: (j // 2, 0, 0)),
        out_shape=jax.ShapeDtypeStruct((max(n // RB // 2, 1), 1, 1),
                                       jnp.float32),
    )


_head_call = pl.pallas_call(
    _head_body,
    out_shape=jax.ShapeDtypeStruct((B, 1), jnp.float32),
)


def kernel(x_p, x_np, y, edge_index_p, edge_index_np, W_gcn, b_gcn,
           W1, b1, W2, b2):
    offs_p = (jnp.arange(C, dtype=jnp.int32) * N_P)[:, None]
    offs_np = (jnp.arange(C, dtype=jnp.int32) * N_NP)[:, None]
    gidx_p = (edge_index_p[0][None, :] + offs_p).reshape(C, GP_P, G)
    gidx_np = (edge_index_np[0][None, :] + offs_np).reshape(C, GP_NP, G)
    dstm_p = edge_index_p[1].reshape(GP_P, G)
    dstm_np = edge_index_np[1].reshape(GP_NP, G)
    zeros = jnp.zeros((N_P // TILES, 16), jnp.float32)
    ones = jnp.ones((G, 16), jnp.float32)
    b2d = b_gcn.reshape(1, D)

    deg_p, deg_np = _deg_call(dstm_p, dstm_np, zeros, ones)

    prep_p = _make_prep(N_P)
    prep_np = _make_prep(N_NP)
    hop_p = _make_hop(N_P)
    hop_np = _make_hop(N_NP)
    trace_p = _make_trace_only(N_P)
    trace_np = _make_trace_only(N_NP)

    gs_p = prep_p(x_p, W_gcn, deg_p)
    gs_np = prep_np(x_np, W_gcn, deg_np)

    trs_p = []
    trs_np = []
    for k in range(WALK_LEN):
        agg_p, agg_np, _, _ = _spmm_call(gs_p, gidx_p, dstm_p,
                                         gs_np, gidx_np, dstm_np)
        if k < WALK_LEN - 1:
            gs_p, tp = hop_p(agg_p, deg_p, b2d, W_gcn)
            gs_np, tn = hop_np(agg_np, deg_np, b2d, W_gcn)
        else:
            tp = trace_p(agg_p, deg_p, b2d)
            tn = trace_np(agg_np, deg_np, b2d)
        trs_p.append(tp.reshape(B))
        trs_np.append(tn.reshape(1))

    tp_all = jnp.stack(trs_p, axis=1)
    tnp_all = jnp.stack(trs_np, axis=1)
    z = _head_call(tp_all, tnp_all, y.astype(jnp.float32),
                   W1, b1.reshape(1, 15), W2, b2.reshape(1, 1))
    return z


# speedup vs baseline: 18.3952x; 1.0121x over previous
import jax
import jax.numpy as jnp
from jax import lax
from jax.experimental import pallas as pl
from jax.experimental.pallas import tpu as pltpu
from jax.experimental.pallas import tpu_sc as plsc

N_P = 32768
N_NP = 4096
D = 128
B = 8
E_P = 524288
E_NP = 65536
WALK_LEN = 7

C = 4
CW = 32
G = 128
TILES = 16
RB = 4096

GP_P = E_P // G
GP_NP = E_NP // G
PGRP_P = GP_P // TILES
PGRP_NP = GP_NP // TILES
SUBP = 16
NBUF = 8


def _deg_body(dst_p, dst_np, zeros, ones,
              deg_p, deg_np,
              acc_p, acc_np, dstv, ones_v, dsem):
    c = lax.axis_index("c")
    s = lax.axis_index("s")
    pltpu.sync_copy(ones, ones_v)
    pltpu.sync_copy(zeros.at[pl.ds(0, N_P // TILES)],
                    acc_p.at[pl.ds(s * (N_P // TILES), N_P // TILES)])
    pltpu.sync_copy(zeros.at[pl.ds(0, N_NP // TILES)],
                    acc_np.at[pl.ds(s * (N_NP // TILES), N_NP // TILES)])
    plsc.subcore_barrier()

    @pl.when(c == 0)
    def _():
        pltpu.sync_copy(dst_p.at[pl.ds(s * PGRP_P, PGRP_P)],
                        dstv.at[pl.ds(0, PGRP_P)])

        def body(j, carry):
            pltpu.async_copy(ones_v, acc_p.at[dstv.at[j]], dsem, add=True)
            return carry
        lax.fori_loop(0, PGRP_P, body, 0)

        def drain(j, carry):
            pltpu.make_async_copy(ones_v, acc_p.at[dstv.at[0]], dsem).wait()
            return carry
        lax.fori_loop(0, PGRP_P, drain, 0)

    @pl.when(c == 1)
    def _():
        pltpu.sync_copy(dst_np.at[pl.ds(s * PGRP_NP, PGRP_NP)],
                        dstv.at[pl.ds(0, PGRP_NP)])

        def body(j, carry):
            pltpu.async_copy(ones_v, acc_np.at[dstv.at[j]], dsem, add=True)
            return carry
        lax.fori_loop(0, PGRP_NP, body, 0)

        def drain(j, carry):
            pltpu.make_async_copy(ones_v, acc_np.at[dstv.at[0]], dsem).wait()
            return carry
        lax.fori_loop(0, PGRP_NP, drain, 0)

    plsc.subcore_barrier()

    @pl.when(c == 0)
    def _():
        pltpu.sync_copy(acc_p.at[pl.ds(s * (N_P // TILES), N_P // TILES)],
                        deg_p.at[pl.ds(s * (N_P // TILES), N_P // TILES)])

    @pl.when(c == 1)
    def _():
        pltpu.sync_copy(acc_np.at[pl.ds(s * (N_NP // TILES), N_NP // TILES)],
                        deg_np.at[pl.ds(s * (N_NP // TILES), N_NP // TILES)])


def _spmm_body(gs_p, gidx_p, dst_p, gs_np, gidx_np, dst_np,
               agg_p, agg_np, tbl_p, tbl_np,
               acc_p, acc_np, rows, gidxv, dstv, gidxnv, dstnv,
               gsem, ssem, tsem, isem, isem2):
    c = lax.axis_index("c")
    s = lax.axis_index("s")
    npt = N_P // TILES
    nnt = N_NP // TILES

    def stage(gs, tbl, accr, nt, chunk, col):
        npieces = nt // G

        def src(pc):
            return gs.at[pl.ds(s * nt + pc * G, G), pl.ds(col, CW)]

        for pc in range(min(NBUF, npieces)):
            pltpu.async_copy(src(pc), rows.at[pc], gsem.at[pc])
        for pc in range(npieces):
            b = pc % NBUF
            pltpu.make_async_copy(src(pc), rows.at[b], gsem.at[b]).wait()
            pltpu.async_copy(
                rows.at[b],
                tbl.at[pl.ds(chunk * nt * TILES + s * nt + pc * G, G)],
                ssem.at[b])
            pltpu.async_copy(rows.at[b], accr.at[pl.ds(s * nt + pc * G, G)],
                             tsem.at[b])
            nxt = pc + NBUF
            if nxt < npieces:
                pltpu.make_async_copy(rows.at[b], tbl.at[pl.ds(0, G)],
                                      ssem.at[b]).wait()
                pltpu.make_async_copy(rows.at[b], accr.at[pl.ds(0, G)],
                                      tsem.at[b]).wait()
                pltpu.async_copy(src(nxt), rows.at[b], gsem.at[b])
        for pc in range(max(0, npieces - NBUF), npieces):
            b = pc % NBUF
            pltpu.make_async_copy(rows.at[b], tbl.at[pl.ds(0, G)],
                                  ssem.at[b]).wait()
            pltpu.make_async_copy(rows.at[b], accr.at[pl.ds(0, G)],
                                  tsem.at[b]).wait()

    def run_stream(table, accr, gv, dv, nsub):
        for b in range(NBUF):
            pltpu.async_copy(table.at[gv.at[b]], rows.at[b], gsem.at[b])

        def it_body(it, carry):
            for b in range(NBUF):
                lj = it * NBUF + b
                pltpu.make_async_copy(table.at[gv.at[lj]], rows.at[b],
                                      gsem.at[b]).wait()
                pltpu.async_copy(rows.at[b], accr.at[dv.at[lj]], ssem.at[b],
                                 add=True)
            for b in range(NBUF):
                lj2 = (it + 1) * NBUF + b

                @pl.when(lj2 < nsub)
                def _():
                    pltpu.make_async_copy(rows.at[b], accr.at[dv.at[b]],
                                          ssem.at[b]).wait()
                    pltpu.async_copy(table.at[gv.at[lj2]], rows.at[b],
                                     gsem.at[b])
            return carry
        lax.fori_loop(0, nsub // NBUF, it_body, 0)
        for b in range(NBUF):
            pltpu.make_async_copy(rows.at[b], accr.at[dv.at[b]],
                                  ssem.at[b]).wait()

    pltpu.sync_copy(dst_np.at[pl.ds(s * PGRP_NP, PGRP_NP)], dstnv)

    for q in range(2):
        for cc in range(2):
            @pl.when(c == cc)
            def _(cc=cc, q=q):
                chunk = cc * 2 + q
                col = chunk * CW
                stage(gs_p, tbl_p, acc_p, npt, chunk, col)
                stage(gs_np, tbl_np, acc_np, nnt, chunk, col)
                pltpu.sync_copy(
                    gidx_np.at[chunk, pl.ds(s * PGRP_NP, PGRP_NP)], gidxnv)
        plsc.subcore_barrier()

        for cc in range(2):
            @pl.when(c == cc)
            def _(cc=cc, q=q):
                chunk = cc * 2 + q
                nsb = PGRP_P // SUBP

                def iload(sj, slot):
                    base = s * PGRP_P + sj * SUBP
                    pltpu.async_copy(gidx_p.at[chunk, pl.ds(base, SUBP)],
                                     gidxv.at[slot], isem)
                    pltpu.async_copy(dst_p.at[pl.ds(base, SUBP)],
                                     dstv.at[slot], isem2)

                def iwait(slot):
                    pltpu.make_async_copy(dst_p.at[pl.ds(0, SUBP)],
                                          gidxv.at[slot], isem).wait()
                    pltpu.make_async_copy(dst_p.at[pl.ds(0, SUBP)],
                                          dstv.at[slot], isem2).wait()

                iload(0, 0)

                def sbody(h, carry):
                    sj0 = h * 2
                    iwait(0)

                    @pl.when(sj0 + 1 < nsb)
                    def _():
                        iload(sj0 + 1, 1)
                    run_stream(tbl_p, acc_p, gidxv.at[0], dstv.at[0], SUBP)
                    iwait(1)

                    @pl.when(sj0 + 2 < nsb)
                    def _():
                        iload(sj0 + 2, 0)
                    run_stream(tbl_p, acc_p, gidxv.at[1], dstv.at[1], SUBP)
                    return carry
                lax.fori_loop(0, nsb // 2, sbody, 0)
                run_stream(tbl_np, acc_np, gidxnv, dstnv, PGRP_NP)
        plsc.subcore_barrier()

        for cc in range(2):
            @pl.when(c == cc)
            def _(cc=cc, q=q):
                col = (cc * 2 + q) * CW
                pltpu.sync_copy(acc_p.at[pl.ds(s * npt, npt)],
                                agg_p.at[pl.ds(s * npt, npt), pl.ds(col, CW)])
                pltpu.sync_copy(acc_np.at[pl.ds(s * nnt, nnt)],
                                agg_np.at[pl.ds(s * nnt, nnt), pl.ds(col, CW)])


_SC_MESH = plsc.VectorSubcoreMesh(core_axis_name="c", subcore_axis_name="s")
_SC_PARAMS = pltpu.CompilerParams(use_tc_tiling_on_sc=False)

_deg_call = pl.kernel(
    _deg_body,
    out_type=(jax.ShapeDtypeStruct((N_P, 16), jnp.float32),
              jax.ShapeDtypeStruct((N_NP, 16), jnp.float32)),
    mesh=_SC_MESH,
    scratch_types=[
        pltpu.VMEM_SHARED((N_P, 16), jnp.float32),
        pltpu.VMEM_SHARED((N_NP, 16), jnp.float32),
        pltpu.VMEM((PGRP_P, G), jnp.int32),
        pltpu.VMEM((G, 16), jnp.float32),
        pltpu.SemaphoreType.DMA,
    ],
    compiler_params=_SC_PARAMS,
)

_spmm_call = pl.kernel(
    _spmm_body,
    out_type=(jax.ShapeDtypeStruct((N_P, D), jnp.float32),
              jax.ShapeDtypeStruct((N_NP, D), jnp.float32),
              jax.ShapeDtypeStruct((C * N_P, CW), jnp.float32),
              jax.ShapeDtypeStruct((C * N_NP, CW), jnp.float32)),
    mesh=_SC_MESH,
    scratch_types=[
        pltpu.VMEM_SHARED((N_P, CW), jnp.float32),
        pltpu.VMEM_SHARED((N_NP, CW), jnp.float32),
        pltpu.VMEM((NBUF, G, CW), jnp.float32),
        pltpu.VMEM((2, SUBP, G), jnp.int32),
        pltpu.VMEM((2, SUBP, G), jnp.int32),
        pltpu.VMEM((PGRP_NP, G), jnp.int32),
        pltpu.VMEM((PGRP_NP, G), jnp.int32),
        pltpu.SemaphoreType.DMA((NBUF,)),
        pltpu.SemaphoreType.DMA((NBUF,)),
        pltpu.SemaphoreType.DMA((NBUF,)),
        pltpu.SemaphoreType.DMA,
        pltpu.SemaphoreType.DMA,
    ],
    compiler_params=_SC_PARAMS,
)


def _prep_body(x_ref, w_ref, deg_ref, out_ref):
    dinv = lax.rsqrt(deg_ref[:, 0:1] + 1.0)
    g = jnp.dot(x_ref[:, :], w_ref[:, :], preferred_element_type=jnp.float32)
    out_ref[:, :] = g * dinv


def _hop_body(agg_ref, deg_ref, b_ref, w_ref, out_ref, tr_ref):
    dinv = lax.rsqrt(deg_ref[:, 0:1] + 1.0)
    h = agg_ref[:, :] * dinv + b_ref[:, :]
    rowi = lax.broadcasted_iota(jnp.int32, (RB, D), 0)
    coli = lax.broadcasted_iota(jnp.int32, (RB, D), 1)
    tr = jnp.sum(jnp.where(rowi == coli, h, 0.0))
    g = jnp.dot(h, w_ref[:, :], preferred_element_type=jnp.float32)
    out_ref[:, :] = g * dinv
    tr_ref[...] = jnp.reshape(tr, (1, 1, 1))


def _trace_only_body(agg_ref, deg_ref, b_ref, tr_ref):
    dinv = lax.rsqrt(deg_ref[:, 0:1] + 1.0)
    h = agg_ref[:, :] * dinv + b_ref[:, :]
    rowi = lax.broadcasted_iota(jnp.int32, (RB, D), 0)
    coli = lax.broadcasted_iota(jnp.int32, (RB, D), 1)
    tr = jnp.sum(jnp.where(rowi == coli, h, 0.0))
    tr_ref[...] = jnp.reshape(tr, (1, 1, 1))


def _head_body(tp_ref, tnp_ref, y_ref, w1_ref, b1_ref, w2_ref, b2_ref, z_ref):
    p = (tp_ref[:, :] - tnp_ref[:, :]) * ((y_ref[:, :] - 0.5) * 2.0)
    mu = jnp.mean(p, axis=0, keepdims=True)
    var = jnp.sum((p - mu) ** 2, axis=0, keepdims=True) / (B - 1)
    p = (p - mu) / jnp.sqrt(var)
    h = jnp.maximum(
        jnp.dot(p, w1_ref[:, :], preferred_element_type=jnp.float32)
        + b1_ref[:, :], 0.0)
    z = (jnp.dot(h, w2_ref[:, :], preferred_element_type=jnp.float32)
         + b2_ref[:, :])
    z_ref[:, :] = 1.0 / (1.0 + jnp.exp(-z))


def _make_prep(n):
    return pl.pallas_call(
        _prep_body,
        grid=(n // RB,),
        in_specs=[
            pl.BlockSpec((RB, D), lambda j: (j, 0)),
            pl.BlockSpec((D, D), lambda j: (0, 0)),
            pl.BlockSpec((RB, 16), lambda j: (j, 0)),
        ],
        out_specs=pl.BlockSpec((RB, D), lambda j: (j, 0)),
        out_shape=jax.ShapeDtypeStruct((n, D), jnp.float32),
    )


def _make_hop(n):
    return pl.pallas_call(
        _hop_body,
        grid=(n // RB,),
        in_specs=[
            pl.BlockSpec((RB, D), lambda j: (j, 0)),
            pl.BlockSpec((RB, 16), lambda j: (j, 0)),
            pl.BlockSpec((1, D), lambda j: (0, 0)),
            pl.BlockSpec((D, D), lambda j: (0, 0)),
        ],
        out_specs=[
            pl.BlockSpec((RB, D), lambda j: (j, 0)),
            pl.BlockSpec((1, 1, 1), lambda j: (j, 0, 0)),
        ],
        out_shape=[
            jax.ShapeDtypeStruct((n, D), jnp.float32),
            jax.ShapeDtypeStruct((n // RB, 1, 1), jnp.float32),
        ],
    )


def _make_trace_only(n):
    return pl.pallas_call(
        _trace_only_body,
        grid=(n // RB,),
        in_specs=[
            pl.BlockSpec((RB, D), lambda j: (j, 0)),
            pl.BlockSpec((RB, 16), lambda j: (j, 0)),
            pl.BlockSpec((1, D), lambda j: (0, 0)),
        ],
        out_specs=pl.BlockSpec((1, 1, 1), lambda j: (j, 0, 0)),
        out_shape=jax.ShapeDtypeStruct((n // RB, 1, 1), jnp.float32),
    )


_head_call = pl.pallas_call(
    _head_body,
    out_shape=jax.ShapeDtypeStruct((B, 1), jnp.float32),
)


def kernel(x_p, x_np, y, edge_index_p, edge_index_np, W_gcn, b_gcn,
           W1, b1, W2, b2):
    offs_p = (jnp.arange(C, dtype=jnp.int32) * N_P)[:, None]
    offs_np = (jnp.arange(C, dtype=jnp.int32) * N_NP)[:, None]
    gidx_p = (edge_index_p[0][None, :] + offs_p).reshape(C, GP_P, G)
    gidx_np = (edge_index_np[0][None, :] + offs_np).reshape(C, GP_NP, G)
    dstm_p = edge_index_p[1].reshape(GP_P, G)
    dstm_np = edge_index_np[1].reshape(GP_NP, G)
    zeros = jnp.zeros((N_P // TILES, 16), jnp.float32)
    ones = jnp.ones((G, 16), jnp.float32)
    b2d = b_gcn.reshape(1, D)

    deg_p, deg_np = _deg_call(dstm_p, dstm_np, zeros, ones)

    prep_p = _make_prep(N_P)
    prep_np = _make_prep(N_NP)
    hop_p = _make_hop(N_P)
    hop_np = _make_hop(N_NP)
    trace_p = _make_trace_only(N_P)
    trace_np = _make_trace_only(N_NP)

    gs_p = prep_p(x_p, W_gcn, deg_p)
    gs_np = prep_np(x_np, W_gcn, deg_np)

    trs_p = []
    trs_np = []
    for k in range(WALK_LEN):
        agg_p, agg_np, _, _ = _spmm_call(gs_p, gidx_p, dstm_p,
                                         gs_np, gidx_np, dstm_np)
        if k < WALK_LEN - 1:
            gs_np, tn = hop_np(agg_np, deg_np, b2d, W_gcn)
            gs_p, tp = hop_p(agg_p, deg_p, b2d, W_gcn)
        else:
            tn = trace_np(agg_np, deg_np, b2d)
            tp = trace_p(agg_p, deg_p, b2d)
        trs_p.append(tp.reshape(B))
        trs_np.append(tn.reshape(1))

    tp_all = jnp.stack(trs_p, axis=1)
    tnp_all = jnp.stack(trs_np, axis=1)
    z = _head_call(tp_all, tnp_all, y.astype(jnp.float32),
                   W1, b1.reshape(1, 15), W2, b2.reshape(1, 1))
    return z
